# baseline (speedup 1.0000x reference)
import functools

import jax
import jax.numpy as jnp
from jax import lax
from jax.experimental import pallas as pl
from jax.experimental.pallas import tpu as pltpu
from jax.experimental.pallas import tpu_sc as plsc

NC = 2
NS = 16
NW = NC * NS
L = 16

N = 10000
NP = 10240
E = 320000
D = 128
DV = D // L
EW = E // NW
CHUNK = 80
NCHUNK = EW // CHUNK
ZROWS = 8
STRIPE = NP // NS
NBUF = 3


def _scale_rows(rows_v, w_ref):
    @plsc.parallel_loop(0, CHUNK // L)
    def scale_group(g):
        wv = w_ref[pl.ds(g * L, L)]
        for k in range(L):
            wb = jnp.take_along_axis(
                wv, jnp.full((L,), k, jnp.int32), axis=0,
                mode="promise_in_bounds")
            e = g * L + k
            for j in range(DV):
                sl = pl.ds(j * L, L)
                rows_v[e, sl] = rows_v[e, sl] * wb


def _sc_spmm(x_hbm, src_hbm, dst_hbm, w_hbm, parts_hbm,
             agg_sh, src_v,
             w_0, w_1, w_2, dst_0, dst_1, dst_2, rows_0, rows_1, rows_2,
             zbuf_v,
             semg_0, semg_1, semg_2, semw_0, semw_1, semw_2,
             semd_0, semd_1, semd_2, sems_0, sems_1, sems_2):
    cid = lax.axis_index("c")
    sid = lax.axis_index("s")
    wid = cid * NS + sid

    rows = (rows_0, rows_1, rows_2)
    wbuf = (w_0, w_1, w_2)
    dst = (dst_0, dst_1, dst_2)
    semg = (semg_0, semg_1, semg_2)
    semw = (semw_0, semw_1, semw_2)
    semd = (semd_0, semd_1, semd_2)
    sems = (sems_0, sems_1, sems_2)

    zeros16 = jnp.zeros((L,), jnp.float32)

    def zrow(i, carry):
        for j in range(DV):
            zbuf_v[i, pl.ds(j * L, L)] = zeros16
        return carry

    lax.fori_loop(0, ZROWS, zrow, 0)
    nz = STRIPE // ZROWS
    for r in range(nz):
        pltpu.async_copy(
            zbuf_v, agg_sh.at[pl.ds(sid * STRIPE + r * ZROWS, ZROWS)], semg_0)
    for r in range(nz):
        pltpu.make_async_copy(
            zbuf_v, agg_sh.at[pl.ds(sid * STRIPE + r * ZROWS, ZROWS)],
            semg_0).wait()

    pltpu.sync_copy(src_hbm.at[wid], src_v)
    zeros16i = jnp.zeros((L,), jnp.int32)
    for g in range(CHUNK // L):
        dst_2[pl.ds(g * L, L)] = zeros16i

    def zrowb(i, carry):
        for j in range(DV):
            rows_2[i, pl.ds(j * L, L)] = zeros16
        return carry

    lax.fori_loop(0, CHUNK, zrowb, 0)
    plsc.subcore_barrier()

    ebase = wid * EW

    def fetch(c, b):
        pltpu.async_copy(x_hbm.at[src_v.at[c]], rows[b], semg[b])
        pltpu.async_copy(
            w_hbm.at[pl.ds(ebase + c * CHUNK, CHUNK)], wbuf[b], semw[b])
        pltpu.async_copy(
            dst_hbm.at[pl.ds(ebase + c * CHUNK, CHUNK)], dst[b], semd[b])

    def wait_fetch(c, b):
        pltpu.make_async_copy(x_hbm.at[src_v.at[c]], rows[b], semg[b]).wait()
        pltpu.make_async_copy(
            w_hbm.at[pl.ds(ebase + c * CHUNK, CHUNK)], wbuf[b], semw[b]).wait()
        pltpu.make_async_copy(
            dst_hbm.at[pl.ds(ebase + c * CHUNK, CHUNK)], dst[b], semd[b]).wait()

    def scatter(b):
        pltpu.async_copy(rows[b], agg_sh.at[dst[b]], sems[b], add=True)

    def wait_scatter(b):
        pltpu.make_async_copy(rows[b], agg_sh.at[dst[b]], sems[b]).wait()

    def step(c, cur, fb, do_fetch=True):
        wait_fetch(c, cur)
        _scale_rows(rows[cur], wbuf[cur])
        wait_scatter(fb)
        if do_fetch:
            fetch(c + 2, fb)
        scatter(cur)

    fetch(0, 0)
    fetch(1, 1)
    scatter(2)

    NB = (NCHUNK - 2) // NBUF

    def pipe_body(i, carry):
        c = NBUF * i
        step(c, 0, 2)
        step(c + 1, 1, 0)
        step(c + 2, 2, 1)
        return carry

    lax.fori_loop(0, NB, pipe_body, 0)
    step(NCHUNK - 2, 0, 2, do_fetch=False)
    step(NCHUNK - 1, 1, 0, do_fetch=False)
    wait_scatter(1)
    plsc.subcore_barrier()

    sl = pl.ds(sid * STRIPE, STRIPE)
    pltpu.sync_copy(agg_sh.at[sl], parts_hbm.at[cid, sl])


_spmm_call = pl.kernel(
    _sc_spmm,
    out_type=jax.ShapeDtypeStruct((NC, NP, D), jnp.float32),
    mesh=plsc.VectorSubcoreMesh(core_axis_name="c", subcore_axis_name="s"),
    scratch_types=[
        pltpu.VMEM_SHARED((NP, D), jnp.float32),
        pltpu.VMEM((NCHUNK, CHUNK), jnp.int32),
        pltpu.VMEM((CHUNK,), jnp.float32),
        pltpu.VMEM((CHUNK,), jnp.float32),
        pltpu.VMEM((CHUNK,), jnp.float32),
        pltpu.VMEM((CHUNK,), jnp.int32),
        pltpu.VMEM((CHUNK,), jnp.int32),
        pltpu.VMEM((CHUNK,), jnp.int32),
        pltpu.VMEM((CHUNK, D), jnp.float32),
        pltpu.VMEM((CHUNK, D), jnp.float32),
        pltpu.VMEM((CHUNK, D), jnp.float32),
        pltpu.VMEM((ZROWS, D), jnp.float32),
    ] + [pltpu.SemaphoreType.DMA] * 12,
)


def _mm_body(p_ref, w_ref, o_ref):
    s = p_ref[0] + p_ref[1]
    o_ref[...] = jnp.maximum(
        jnp.dot(s, w_ref[...], preferred_element_type=jnp.float32), 0.0)


_MM_BLOCK = 1000

_mm_call = pl.pallas_call(
    _mm_body,
    grid=(N // _MM_BLOCK,),
    in_specs=[
        pl.BlockSpec((NC, _MM_BLOCK, D), lambda i: (0, i, 0)),
        pl.BlockSpec((D, D), lambda i: (0, 0)),
    ],
    out_specs=pl.BlockSpec((_MM_BLOCK, D), lambda i: (i, 0)),
    out_shape=jax.ShapeDtypeStruct((N, D), jnp.float32),
)


@jax.jit
def kernel(input, edge_index, edge_weight, W):
    src = edge_index[0].reshape(NW, NCHUNK, CHUNK)
    dst = edge_index[1]
    parts = _spmm_call(input, src, dst, edge_weight)
    return _mm_call(parts, W)


# speedup vs baseline: 1.0136x; 1.0136x over previous
import functools

import jax
import jax.numpy as jnp
from jax import lax
from jax.experimental import pallas as pl
from jax.experimental.pallas import tpu as pltpu
from jax.experimental.pallas import tpu_sc as plsc

NC = 2
NS = 16
NW = NC * NS
L = 16

N = 10000
NP = 10240
E = 320000
D = 128
DV = D // L
EW = E // NW
CHUNK = 80
NCHUNK = EW // CHUNK
ZROWS = 8
STRIPE = NP // NS
NBUF = 3


def _scale_rows(rows_v, w_ref):
    @plsc.parallel_loop(0, CHUNK // L)
    def scale_group(g):
        wv = w_ref[pl.ds(g * L, L)]
        for k in range(L):
            wb = jnp.take_along_axis(
                wv, jnp.full((L,), k, jnp.int32), axis=0,
                mode="promise_in_bounds")
            e = g * L + k
            for j in range(DV):
                sl = pl.ds(j * L, L)
                rows_v[e, sl] = rows_v[e, sl] * wb


def _sc_spmm(x_hbm, src_hbm, dst_hbm, w_hbm, parts_hbm,
             agg_sh, src_v,
             w_0, w_1, w_2, dst_0, dst_1, dst_2, rows_0, rows_1, rows_2,
             zbuf_v,
             semg_0, semg_1, semg_2, semw_0, semw_1, semw_2,
             semd_0, semd_1, semd_2, sems_0, sems_1, sems_2):
    cid = lax.axis_index("c")
    sid = lax.axis_index("s")
    wid = cid * NS + sid

    rows = (rows_0, rows_1, rows_2)
    wbuf = (w_0, w_1, w_2)
    dst = (dst_0, dst_1, dst_2)
    semg = (semg_0, semg_1, semg_2)
    semw = (semw_0, semw_1, semw_2)
    semd = (semd_0, semd_1, semd_2)
    sems = (sems_0, sems_1, sems_2)

    zeros16 = jnp.zeros((L,), jnp.float32)

    def zrow(i, carry):
        for j in range(DV):
            zbuf_v[i, pl.ds(j * L, L)] = zeros16
        return carry

    lax.fori_loop(0, ZROWS, zrow, 0)
    nz = STRIPE // ZROWS
    for r in range(nz):
        pltpu.async_copy(
            zbuf_v, agg_sh.at[pl.ds(sid * STRIPE + r * ZROWS, ZROWS)], semg_0)
    for r in range(nz):
        pltpu.make_async_copy(
            zbuf_v, agg_sh.at[pl.ds(sid * STRIPE + r * ZROWS, ZROWS)],
            semg_0).wait()

    pltpu.sync_copy(src_hbm.at[wid], src_v)
    zeros16i = jnp.zeros((L,), jnp.int32)
    for g in range(CHUNK // L):
        dst_2[pl.ds(g * L, L)] = zeros16i

    def zrowb(i, carry):
        for j in range(DV):
            rows_2[i, pl.ds(j * L, L)] = zeros16
        return carry

    lax.fori_loop(0, CHUNK, zrowb, 0)
    plsc.subcore_barrier()

    ebase = wid * EW

    def fetch(c, b):
        pltpu.async_copy(x_hbm.at[src_v.at[c]], rows[b], semg[b])
        pltpu.async_copy(
            w_hbm.at[pl.ds(ebase + c * CHUNK, CHUNK)], wbuf[b], semw[b])
        pltpu.async_copy(
            dst_hbm.at[pl.ds(ebase + c * CHUNK, CHUNK)], dst[b], semd[b])

    def wait_fetch(c, b):
        pltpu.make_async_copy(x_hbm.at[src_v.at[c]], rows[b], semg[b]).wait()
        pltpu.make_async_copy(
            w_hbm.at[pl.ds(ebase + c * CHUNK, CHUNK)], wbuf[b], semw[b]).wait()
        pltpu.make_async_copy(
            dst_hbm.at[pl.ds(ebase + c * CHUNK, CHUNK)], dst[b], semd[b]).wait()

    def scatter(b):
        pltpu.async_copy(rows[b], agg_sh.at[dst[b]], sems[b], add=True)

    def wait_scatter(b):
        pltpu.make_async_copy(rows[b], agg_sh.at[dst[b]], sems[b]).wait()

    def step(c, cur, fb, do_fetch=True):
        wait_fetch(c, cur)
        _scale_rows(rows[cur], wbuf[cur])
        wait_scatter(fb)
        if do_fetch:
            fetch(c + 2, fb)
        scatter(cur)

    fetch(0, 0)
    fetch(1, 1)
    scatter(2)

    NB = (NCHUNK - 2) // NBUF

    def pipe_body(i, carry):
        c = NBUF * i
        step(c, 0, 2)
        step(c + 1, 1, 0)
        step(c + 2, 2, 1)
        return carry

    lax.fori_loop(0, NB, pipe_body, 0)
    step(NCHUNK - 2, 0, 2, do_fetch=False)
    step(NCHUNK - 1, 1, 0, do_fetch=False)
    wait_scatter(1)
    plsc.subcore_barrier()

    sl = pl.ds(sid * STRIPE, STRIPE)
    pltpu.sync_copy(agg_sh.at[sl], parts_hbm.at[cid, sl])


_spmm_call = pl.kernel(
    _sc_spmm,
    out_type=jax.ShapeDtypeStruct((NC, NP, D), jnp.float32),
    mesh=plsc.VectorSubcoreMesh(core_axis_name="c", subcore_axis_name="s"),
    scratch_types=[
        pltpu.VMEM_SHARED((NP, D), jnp.float32),
        pltpu.VMEM((NCHUNK, CHUNK), jnp.int32),
        pltpu.VMEM((CHUNK,), jnp.float32),
        pltpu.VMEM((CHUNK,), jnp.float32),
        pltpu.VMEM((CHUNK,), jnp.float32),
        pltpu.VMEM((CHUNK,), jnp.int32),
        pltpu.VMEM((CHUNK,), jnp.int32),
        pltpu.VMEM((CHUNK,), jnp.int32),
        pltpu.VMEM((CHUNK, D), jnp.float32),
        pltpu.VMEM((CHUNK, D), jnp.float32),
        pltpu.VMEM((CHUNK, D), jnp.float32),
        pltpu.VMEM((ZROWS, D), jnp.float32),
    ] + [pltpu.SemaphoreType.DMA] * 12,
)


def _mm_body(p_ref, w_ref, o_ref):
    s = p_ref[0] + p_ref[1]
    o_ref[...] = jnp.maximum(
        jnp.dot(s, w_ref[...], preferred_element_type=jnp.float32), 0.0)


_MM_BLOCK = 2000

_mm_call = pl.pallas_call(
    _mm_body,
    grid=(N // _MM_BLOCK,),
    in_specs=[
        pl.BlockSpec((NC, _MM_BLOCK, D), lambda i: (0, i, 0)),
        pl.BlockSpec((D, D), lambda i: (0, 0)),
    ],
    out_specs=pl.BlockSpec((_MM_BLOCK, D), lambda i: (i, 0)),
    out_shape=jax.ShapeDtypeStruct((N, D), jnp.float32),
)


@jax.jit
def kernel(input, edge_index, edge_weight, W):
    src = edge_index[0].reshape(NW, NCHUNK, CHUNK)
    dst = edge_index[1]
    parts = _spmm_call(input, src, dst, edge_weight)
    return _mm_call(parts, W)
